# Initial kernel scaffold; baseline (speedup 1.0000x reference)
import jax
import jax.numpy as jnp
from jax.experimental import pallas as pl


def kernel(hidden_states, W, b, codevectors):
    raise NotImplementedError("write your pallas kernel here")


# speedup vs baseline: 8.9657x; 8.9657x over previous
import functools

import jax
import jax.numpy as jnp
from jax import lax
from jax.experimental import pallas as pl

_T = 512


def _vq_body(x_ref, w_ref, b_ref, cv_ref,
             dist_ref, enc_ref, quant_ref, counts_ref, perp_ref,
             *, n_tokens, n_steps, n_groups, n_codes, group_dim):
    i = pl.program_id(0)

    @pl.when(i == 0)
    def _init():
        counts_ref[...] = jnp.zeros_like(counts_ref)

    x = x_ref[...]
    w = w_ref[...]
    b = b_ref[...]
    h = jnp.dot(x, w, preferred_element_type=jnp.float32) + b

    iota = lax.broadcasted_iota(jnp.int32, (x.shape[0], n_codes), 1)
    count_rows = []
    for g in range(n_groups):
        hg = h[:, g * group_dim:(g + 1) * group_dim]
        cvg = cv_ref[g]
        hc = lax.dot_general(
            hg, cvg, (((1,), (1,)), ((), ())),
            preferred_element_type=jnp.float32,
            precision=lax.Precision.HIGHEST)
        h2 = jnp.sum(hg * hg, axis=1, keepdims=True)
        c2 = jnp.sum(cvg * cvg, axis=1)[None, :]
        dist = h2 - 2.0 * hc + c2
        dist_ref[g] = dist
        mn = jnp.min(dist, axis=1, keepdims=True)
        idx = jnp.min(jnp.where(dist == mn, iota, n_codes), axis=1)
        enc = (iota == idx[:, None]).astype(jnp.float32)
        enc_ref[g] = enc
        quant_ref[:, g * group_dim:(g + 1) * group_dim] = jnp.dot(
            enc, cvg, preferred_element_type=jnp.float32,
            precision=lax.Precision.HIGHEST)
        count_rows.append(jnp.sum(enc, axis=0, keepdims=True))

    counts_ref[...] = counts_ref[...] + jnp.concatenate(count_rows, axis=0)

    @pl.when(i == n_steps - 1)
    def _final():
        probs = counts_ref[...] * (1.0 / n_tokens)
        probs = jnp.clip(probs, 1e-10, 1.0)
        ent = -jnp.sum(probs * jnp.log(probs + 1e-10), axis=1, keepdims=True)
        perp_ref[...] = jnp.mean(jnp.exp(ent), axis=0, keepdims=True)


def kernel(hidden_states, W, b, codevectors):
    B, S, D = hidden_states.shape
    G, K, GD = codevectors.shape
    N = B * S
    assert N % _T == 0
    n_steps = N // _T

    x = hidden_states.reshape(N, D)
    b2 = b.reshape(1, G * GD)

    body = functools.partial(
        _vq_body, n_tokens=N, n_steps=n_steps, n_groups=G,
        n_codes=K, group_dim=GD)

    dist, enc, quant, _counts, perp = pl.pallas_call(
        body,
        grid=(n_steps,),
        in_specs=[
            pl.BlockSpec((_T, D), lambda i: (i, 0)),
            pl.BlockSpec((D, G * GD), lambda i: (0, 0)),
            pl.BlockSpec((1, G * GD), lambda i: (0, 0)),
            pl.BlockSpec((G, K, GD), lambda i: (0, 0, 0)),
        ],
        out_specs=[
            pl.BlockSpec((G, _T, K), lambda i: (0, i, 0)),
            pl.BlockSpec((G, _T, K), lambda i: (0, i, 0)),
            pl.BlockSpec((_T, G * GD), lambda i: (i, 0)),
            pl.BlockSpec((G, K), lambda i: (0, 0)),
            pl.BlockSpec((1, 1), lambda i: (0, 0)),
        ],
        out_shape=[
            jax.ShapeDtypeStruct((G, N, K), jnp.float32),
            jax.ShapeDtypeStruct((G, N, K), jnp.float32),
            jax.ShapeDtypeStruct((N, G * GD), jnp.float32),
            jax.ShapeDtypeStruct((G, K), jnp.float32),
            jax.ShapeDtypeStruct((1, 1), jnp.float32),
        ],
    )(x, W, b2, codevectors)

    quantized_features = quant.reshape(B, S, G * GD)
    encodings = enc.reshape(G, B, S, K)
    distances = dist.reshape(G, B, S, K)
    return (quantized_features, encodings, distances, perp.reshape(()))


# speedup vs baseline: 9.7564x; 1.0882x over previous
import functools

import jax
import jax.numpy as jnp
from jax import lax
from jax.experimental import pallas as pl
from jax.experimental.pallas import tpu as pltpu

_T = 512


def _vq_body(x_ref, w_ref, b_ref, cv_ref,
             dist_ref, enc_ref, quant_ref, pcounts_ref,
             *, n_groups, n_codes, group_dim):
    x = x_ref[...]
    w = w_ref[...]
    b = b_ref[...]
    h = jnp.dot(x, w, preferred_element_type=jnp.float32) + b

    iota = lax.broadcasted_iota(jnp.int32, (x.shape[0], n_codes), 1)
    count_rows = []
    for g in range(n_groups):
        hg = h[:, g * group_dim:(g + 1) * group_dim]
        cvg = cv_ref[g]
        hc = lax.dot_general(
            hg, cvg, (((1,), (1,)), ((), ())),
            preferred_element_type=jnp.float32,
            precision=lax.Precision.HIGHEST)
        h2 = jnp.sum(hg * hg, axis=1, keepdims=True)
        c2 = jnp.sum(cvg * cvg, axis=1)[None, :]
        dist = h2 - 2.0 * hc + c2
        dist_ref[g] = dist
        mn = jnp.min(dist, axis=1, keepdims=True)
        idx = jnp.min(jnp.where(dist == mn, iota, n_codes), axis=1)
        enc = (iota == idx[:, None]).astype(jnp.float32)
        enc_ref[g] = enc
        enc16 = enc.astype(jnp.bfloat16)
        cv_hi = cvg.astype(jnp.bfloat16)
        cv_lo = (cvg - cv_hi.astype(jnp.float32)).astype(jnp.bfloat16)
        quant_ref[:, g * group_dim:(g + 1) * group_dim] = (
            jnp.dot(enc16, cv_hi, preferred_element_type=jnp.float32)
            + jnp.dot(enc16, cv_lo, preferred_element_type=jnp.float32))
        count_rows.append(jnp.sum(enc, axis=0, keepdims=True))

    pcounts_ref[0] = jnp.concatenate(count_rows, axis=0)


def _perp_body(pcounts_ref, perp_ref, *, n_tokens):
    counts = jnp.sum(pcounts_ref[...], axis=0)
    probs = counts * (1.0 / n_tokens)
    probs = jnp.clip(probs, 1e-10, 1.0)
    ent = -jnp.sum(probs * jnp.log(probs + 1e-10), axis=1, keepdims=True)
    perp_ref[...] = jnp.mean(jnp.exp(ent), axis=0, keepdims=True)


def kernel(hidden_states, W, b, codevectors):
    B, S, D = hidden_states.shape
    G, K, GD = codevectors.shape
    N = B * S
    assert N % _T == 0
    n_steps = N // _T

    x = hidden_states.reshape(N, D)
    b2 = b.reshape(1, G * GD)

    body = functools.partial(
        _vq_body, n_groups=G, n_codes=K, group_dim=GD)

    dist, enc, quant, pcounts = pl.pallas_call(
        body,
        grid=(n_steps,),
        in_specs=[
            pl.BlockSpec((_T, D), lambda i: (i, 0)),
            pl.BlockSpec((D, G * GD), lambda i: (0, 0)),
            pl.BlockSpec((1, G * GD), lambda i: (0, 0)),
            pl.BlockSpec((G, K, GD), lambda i: (0, 0, 0)),
        ],
        out_specs=[
            pl.BlockSpec((G, _T, K), lambda i: (0, i, 0)),
            pl.BlockSpec((G, _T, K), lambda i: (0, i, 0)),
            pl.BlockSpec((_T, G * GD), lambda i: (i, 0)),
            pl.BlockSpec((1, G, K), lambda i: (i, 0, 0)),
        ],
        out_shape=[
            jax.ShapeDtypeStruct((G, N, K), jnp.float32),
            jax.ShapeDtypeStruct((G, N, K), jnp.float32),
            jax.ShapeDtypeStruct((N, G * GD), jnp.float32),
            jax.ShapeDtypeStruct((n_steps, G, K), jnp.float32),
        ],
        compiler_params=pltpu.CompilerParams(
            dimension_semantics=("parallel",)),
    )(x, W, b2, codevectors)

    perp = pl.pallas_call(
        functools.partial(_perp_body, n_tokens=N),
        out_shape=jax.ShapeDtypeStruct((1, 1), jnp.float32),
    )(pcounts)

    quantized_features = quant.reshape(B, S, G * GD)
    encodings = enc.reshape(G, B, S, K)
    distances = dist.reshape(G, B, S, K)
    return (quantized_features, encodings, distances, perp.reshape(()))


# speedup vs baseline: 9.8277x; 1.0073x over previous
import functools

import jax
import jax.numpy as jnp
from jax import lax
from jax.experimental import pallas as pl
from jax.experimental.pallas import tpu as pltpu

_T = 512


def _vq_body(x_ref, w_ref, b_ref, cv_ref,
             quant_ref, enc_ref, dist_ref, pcounts_ref,
             *, n_groups, n_codes, group_dim):
    x = x_ref[0]
    w = w_ref[...]
    b = b_ref[...]
    h = jnp.dot(x, w, preferred_element_type=jnp.float32) + b

    iota = lax.broadcasted_iota(jnp.int32, (x.shape[0], n_codes), 1)
    count_rows = []
    for g in range(n_groups):
        hg = h[:, g * group_dim:(g + 1) * group_dim]
        cvg = cv_ref[g]
        hc = lax.dot_general(
            hg, cvg, (((1,), (1,)), ((), ())),
            preferred_element_type=jnp.float32,
            precision=lax.Precision.HIGHEST)
        h2 = jnp.sum(hg * hg, axis=1, keepdims=True)
        c2 = jnp.sum(cvg * cvg, axis=1)[None, :]
        dist = h2 - 2.0 * hc + c2
        dist_ref[g, 0] = dist
        mn = jnp.min(dist, axis=1, keepdims=True)
        idx = jnp.min(jnp.where(dist == mn, iota, n_codes), axis=1)
        enc = (iota == idx[:, None]).astype(jnp.float32)
        enc_ref[g, 0] = enc
        enc16 = enc.astype(jnp.bfloat16)
        cv_hi = cvg.astype(jnp.bfloat16)
        cv_lo = (cvg - cv_hi.astype(jnp.float32)).astype(jnp.bfloat16)
        quant_ref[0, :, g * group_dim:(g + 1) * group_dim] = (
            jnp.dot(enc16, cv_hi, preferred_element_type=jnp.float32)
            + jnp.dot(enc16, cv_lo, preferred_element_type=jnp.float32))
        count_rows.append(jnp.sum(enc, axis=0, keepdims=True))

    pcounts_ref[0] = jnp.concatenate(count_rows, axis=0)


def _perp_body(pcounts_ref, perp_ref, *, n_tokens):
    counts = jnp.sum(pcounts_ref[...], axis=0)
    probs = counts * (1.0 / n_tokens)
    probs = jnp.clip(probs, 1e-10, 1.0)
    ent = -jnp.sum(probs * jnp.log(probs + 1e-10), axis=1, keepdims=True)
    perp_ref[...] = jnp.mean(jnp.exp(ent), axis=0, keepdims=True)


def kernel(hidden_states, W, b, codevectors):
    B, S, D = hidden_states.shape
    G, K, GD = codevectors.shape
    assert S % _T == 0
    sblocks = S // _T
    n_steps = B * sblocks

    b2 = b.reshape(1, G * GD)

    body = functools.partial(
        _vq_body, n_groups=G, n_codes=K, group_dim=GD)

    quant, enc, dist, pcounts = pl.pallas_call(
        body,
        grid=(B, sblocks),
        in_specs=[
            pl.BlockSpec((1, _T, D), lambda i, j: (i, j, 0)),
            pl.BlockSpec((D, G * GD), lambda i, j: (0, 0)),
            pl.BlockSpec((1, G * GD), lambda i, j: (0, 0)),
            pl.BlockSpec((G, K, GD), lambda i, j: (0, 0, 0)),
        ],
        out_specs=[
            pl.BlockSpec((1, _T, G * GD), lambda i, j: (i, j, 0)),
            pl.BlockSpec((G, 1, _T, K), lambda i, j: (0, i, j, 0)),
            pl.BlockSpec((G, 1, _T, K), lambda i, j: (0, i, j, 0)),
            pl.BlockSpec((1, G, K), lambda i, j: (i * (S // _T) + j, 0, 0)),
        ],
        out_shape=[
            jax.ShapeDtypeStruct((B, S, G * GD), jnp.float32),
            jax.ShapeDtypeStruct((G, B, S, K), jnp.float32),
            jax.ShapeDtypeStruct((G, B, S, K), jnp.float32),
            jax.ShapeDtypeStruct((n_steps, G, K), jnp.float32),
        ],
        compiler_params=pltpu.CompilerParams(
            dimension_semantics=("parallel", "parallel")),
    )(hidden_states, W, b2, codevectors)

    perp = pl.pallas_call(
        functools.partial(_perp_body, n_tokens=B * S),
        out_shape=jax.ShapeDtypeStruct((1, 1), jnp.float32),
    )(pcounts)

    return (quant, enc, dist, perp.reshape(()))


# speedup vs baseline: 24.6955x; 2.5128x over previous
import functools

import jax
import jax.numpy as jnp
from jax import lax
from jax.experimental import pallas as pl
from jax.experimental.pallas import tpu as pltpu


def _vq_body(x_ref, w_ref, b_ref, cv_ref,
             quant_ref, enc_ref, dist_ref, pcounts_ref,
             *, n_groups, n_codes, group_dim):
    x = x_ref[0]
    w = w_ref[...]
    b = b_ref[...]
    h = lax.dot_general(w, x, (((0,), (1,)), ((), ())),
                        preferred_element_type=jnp.float32) + b

    seq = x.shape[0]
    iota = lax.broadcasted_iota(jnp.int32, (n_codes, seq), 0)
    ones_row = jnp.ones((1, seq), dtype=jnp.bfloat16)
    count_rows = []
    for g in range(n_groups):
        hg = h[g * group_dim:(g + 1) * group_dim, :]
        cvg = cv_ref[g]
        hc = lax.dot_general(
            cvg, hg, (((1,), (0,)), ((), ())),
            preferred_element_type=jnp.float32,
            precision=lax.Precision.HIGHEST)
        h2 = jnp.sum(hg * hg, axis=0, keepdims=True)
        c2 = jnp.sum(cvg * cvg, axis=1, keepdims=True)
        dist = h2 - 2.0 * hc + c2
        dist_ref[g, 0] = dist
        mn = jnp.min(dist, axis=0, keepdims=True)
        idx = jnp.min(jnp.where(dist == mn, iota, n_codes),
                      axis=0, keepdims=True)
        enc = (iota == idx).astype(jnp.float32)
        enc_ref[g, 0] = enc
        enc16 = enc.astype(jnp.bfloat16)
        cv_hi = cvg.astype(jnp.bfloat16)
        cv_lo = (cvg - cv_hi.astype(jnp.float32)).astype(jnp.bfloat16)
        quant_ref[0, :, g * group_dim:(g + 1) * group_dim] = (
            lax.dot_general(enc16, cv_hi, (((0,), (0,)), ((), ())),
                            preferred_element_type=jnp.float32)
            + lax.dot_general(enc16, cv_lo, (((0,), (0,)), ((), ())),
                              preferred_element_type=jnp.float32))
        count_rows.append(
            lax.dot_general(ones_row, enc16, (((1,), (1,)), ((), ())),
                            preferred_element_type=jnp.float32))

    pcounts_ref[0] = jnp.concatenate(count_rows, axis=0)


def _perp_body(pcounts_ref, perp_ref, *, n_tokens):
    counts = jnp.sum(pcounts_ref[...], axis=0)
    probs = counts * (1.0 / n_tokens)
    probs = jnp.clip(probs, 1e-10, 1.0)
    ent = -jnp.sum(probs * jnp.log(probs + 1e-10), axis=1, keepdims=True)
    perp_ref[...] = jnp.mean(jnp.exp(ent), axis=0, keepdims=True)


def kernel(hidden_states, W, b, codevectors):
    B, S, D = hidden_states.shape
    G, K, GD = codevectors.shape
    C = G * GD

    b2 = b.reshape(C, 1)

    body = functools.partial(
        _vq_body, n_groups=G, n_codes=K, group_dim=GD)

    quant, enc_t, dist_t, pcounts = pl.pallas_call(
        body,
        grid=(B,),
        in_specs=[
            pl.BlockSpec((1, S, D), lambda i: (i, 0, 0)),
            pl.BlockSpec((D, C), lambda i: (0, 0)),
            pl.BlockSpec((C, 1), lambda i: (0, 0)),
            pl.BlockSpec((G, K, GD), lambda i: (0, 0, 0)),
        ],
        out_specs=[
            pl.BlockSpec((1, S, C), lambda i: (i, 0, 0)),
            pl.BlockSpec((G, 1, K, S), lambda i: (0, i, 0, 0)),
            pl.BlockSpec((G, 1, K, S), lambda i: (0, i, 0, 0)),
            pl.BlockSpec((1, G, K), lambda i: (i, 0, 0)),
        ],
        out_shape=[
            jax.ShapeDtypeStruct((B, S, C), jnp.float32),
            jax.ShapeDtypeStruct((G, B, K, S), jnp.float32),
            jax.ShapeDtypeStruct((G, B, K, S), jnp.float32),
            jax.ShapeDtypeStruct((B, G, K), jnp.float32),
        ],
        compiler_params=pltpu.CompilerParams(
            dimension_semantics=("parallel",)),
    )(hidden_states, W, b2, codevectors)

    perp = pl.pallas_call(
        functools.partial(_perp_body, n_tokens=B * S),
        out_shape=jax.ShapeDtypeStruct((1, 1), jnp.float32),
    )(pcounts)

    encodings = jnp.transpose(enc_t, (0, 1, 3, 2))
    distances = jnp.transpose(dist_t, (0, 1, 3, 2))
    return (quant, encodings, distances, perp.reshape(()))
